# baseline (speedup 1.0000x reference)
import functools
import jax
import jax.numpy as jnp
from jax import lax
from jax.experimental import pallas as pl
from jax.experimental.pallas import tpu as pltpu
from jax.experimental.pallas import tpu_sc as plsc

NC = 2
NS = 16
NW = NC * NS
LANES = 16

B = 16384
L = 50
D = 64
V = 1000000

G = 128
BPW = B // NW
QPW = BPW // G
UNITS = L * QPW
NBLK = B // G
DBLK = D // 8
NGRP = G // LANES


def _embed_body(x_hbm, xlen_hbm, w_hbm, out5_hbm, ret4_hbm,
                idx0, idx1, gidx0, gidx1, pb0, pb1,
                pairs0, pairs1, stage0, stage1,
                acc_v, inv_v,
                sem_g0, sem_g1, sem_o0, sem_o1):
    idx = (idx0, idx1)
    gidx = (gidx0, gidx1)
    pb = (pb0, pb1)
    pairs = (pairs0, pairs1)
    stage = (stage0, stage1)
    sem_g = (sem_g0, sem_g1)
    sem_o = (sem_o0, sem_o1)

    wid = lax.axis_index("s") * NC + lax.axis_index("c")
    b0 = wid * BPW

    pltpu.sync_copy(xlen_hbm.at[pl.ds(pl.multiple_of(b0, BPW), BPW)], inv_v)
    for k in range(BPW // LANES):
        inv_v[pl.ds(k * LANES, LANES)] = 1.0 / inv_v[pl.ds(k * LANES, LANES)]

    zero = jnp.zeros((LANES,), jnp.float32)

    def zero_body(t, carry):
        dblk = t // QPW
        bb = t % QPW
        for sub in range(8):
            for g in range(NGRP):
                acc_v[dblk, bb, sub, pl.ds(g * LANES, LANES)] = zero
        return carry

    lax.fori_loop(0, DBLK * QPW, zero_body, 0)

    def stage_idx(u, b):
        l = u // QPW
        q = u % QPW
        off = pl.multiple_of(l * B + b0 + q * G, G)
        pltpu.sync_copy(x_hbm.at[pl.ds(off, G)], idx[b])
        for k in range(NGRP):
            xv = idx[b][pl.ds(k * LANES, LANES)]
            gidx[b][pl.ds(k * LANES, LANES)] = lax.shift_right_logical(xv, 1)
            pb[b][pl.ds(k * LANES, LANES)] = lax.shift_left(xv & 1, 6)
        pltpu.make_async_copy(w_hbm.at[gidx[b]], pairs[b], sem_g[b]).start()

    def drain_gather(b):
        pltpu.make_async_copy(w_hbm.at[pl.ds(0, G)], pairs[b], sem_g[b]).wait()

    def out_copy(u, b):
        l = u // QPW
        bb = wid * QPW + (u % QPW)
        return pltpu.make_async_copy(
            stage[b], out5_hbm.at[l, :, pl.ds(bb, 1)], sem_o[b])

    def drain_out(b):
        pltpu.make_async_copy(stage[b], out5_hbm.at[0, :, pl.ds(0, 1)],
                              sem_o[b]).wait()

    stage_idx(0, 0)

    lane = lax.iota(jnp.int32, LANES)

    def unit_body(uu, carry):
        for b in range(2):
            u = uu * 2 + b
            b2 = 1 - b
            drain_gather(b)

            @pl.when(u + 1 < UNITS)
            def _prefetch():
                stage_idx(u + 1, b2)

            @pl.when(u >= 2)
            def _():
                drain_out(b)

            q = u % QPW
            for g in range(NGRP):
                slot16 = lane + (g * LANES)
                pb16 = pb[b][pl.ds(g * LANES, LANES)]

                def sel_body(dblk, carry2):
                    for sub in range(8):
                        d = dblk * 8 + sub
                        v = plsc.load_gather(pairs[b], [slot16, pb16 + d])
                        stage[b][dblk, 0, sub, pl.ds(g * LANES, LANES)] = v
                        plsc.addupdate(
                            acc_v.at[dblk, q, sub, pl.ds(g * LANES, LANES)], v)
                    return carry2

                lax.fori_loop(0, DBLK, sel_body, 0)

            out_copy(u, b).start()
        return carry

    lax.fori_loop(0, UNITS // 2, unit_body, 0)
    drain_out(0)
    drain_out(1)

    def scale_body(t, carry):
        dblk = t // QPW
        bb = t % QPW
        for sub in range(8):
            for g in range(NGRP):
                sl = pl.ds(g * LANES, LANES)
                iv = inv_v[pl.ds(bb * G + g * LANES, LANES)]
                acc_v[dblk, bb, sub, sl] = acc_v[dblk, bb, sub, sl] * iv
        return carry

    lax.fori_loop(0, DBLK * QPW, scale_body, 0)
    pltpu.sync_copy(
        acc_v, ret4_hbm.at[:, pl.ds(pl.multiple_of(wid * QPW, QPW), QPW)])


_embed_kernel = functools.partial(
    pl.kernel,
    out_type=(jax.ShapeDtypeStruct((L, DBLK, NBLK, 8, G), jnp.float32),
              jax.ShapeDtypeStruct((DBLK, NBLK, 8, G), jnp.float32)),
    mesh=plsc.VectorSubcoreMesh(core_axis_name="c", subcore_axis_name="s"),
    compiler_params=pltpu.CompilerParams(use_tc_tiling_on_sc=False,
                                         needs_layout_passes=False),
    scratch_types=[
        pltpu.VMEM((G,), jnp.int32),
        pltpu.VMEM((G,), jnp.int32),
        pltpu.VMEM((G,), jnp.int32),
        pltpu.VMEM((G,), jnp.int32),
        pltpu.VMEM((G,), jnp.int32),
        pltpu.VMEM((G,), jnp.int32),
        pltpu.VMEM((G, 128), jnp.float32),
        pltpu.VMEM((G, 128), jnp.float32),
        pltpu.VMEM((DBLK, 1, 8, G), jnp.float32),
        pltpu.VMEM((DBLK, 1, 8, G), jnp.float32),
        pltpu.VMEM((DBLK, QPW, 8, G), jnp.float32),
        pltpu.VMEM((BPW,), jnp.float32),
        pltpu.SemaphoreType.DMA,
        pltpu.SemaphoreType.DMA,
        pltpu.SemaphoreType.DMA,
        pltpu.SemaphoreType.DMA,
    ],
)(_embed_body)


def kernel(x, x_len, weight):
    x1d = x.T.reshape(B * L).astype(jnp.int32)
    xlen = x_len.reshape(B).astype(jnp.float32)
    w128 = weight.reshape(V // 2, 128)
    out5, ret4 = _embed_kernel(x1d, xlen, w128)
    raw = jnp.transpose(out5, (2, 4, 0, 1, 3)).reshape(B, L, D)
    ret = jnp.transpose(ret4, (1, 3, 0, 2)).reshape(B, D)
    return (ret, raw)
